# Initial kernel scaffold; baseline (speedup 1.0000x reference)
import jax
import jax.numpy as jnp
from jax.experimental import pallas as pl


def kernel(input_ids, token_type_ids, word_table, pos_table, type_table, ln_gamma, ln_beta, W, b):
    raise NotImplementedError("write your pallas kernel here")


# speedup vs baseline: 1.9379x; 1.9379x over previous
import functools

import jax
import jax.numpy as jnp
from jax import lax
from jax.experimental import pallas as pl
from jax.experimental.pallas import tpu as pltpu

try:
    from jax.experimental.pallas import tpu_sc as plsc
    _INFO = plsc.get_sparse_core_info()
    _NC, _NS = _INFO.num_cores, _INFO.num_subcores
except Exception:
    plsc = None
    _NC, _NS = 2, 16

_NW = _NC * _NS
_S = 8192
_D = 768
_EPS = 1e-12
_CH = 64
_BS = 512


def _sc_gather(word_table, idx3):
    nchunk = idx3.shape[1]
    b_per_w = nchunk * _CH
    mesh = plsc.VectorSubcoreMesh(core_axis_name="c", subcore_axis_name="s")

    @functools.partial(
        pl.kernel,
        mesh=mesh,
        out_type=jax.ShapeDtypeStruct((_S, _D), jnp.float32),
        scratch_types=[
            pltpu.VMEM((nchunk, _CH), jnp.int32),
            pltpu.VMEM((_CH, _D), jnp.float32),
            pltpu.VMEM((_CH, _D), jnp.float32),
            pltpu.SemaphoreType.DMA,
            pltpu.SemaphoreType.DMA,
        ],
    )
    def k(table_hbm, idx_hbm, out_hbm, idx_v, buf0, buf1, sem0, sem1):
        wid = lax.axis_index("s") * _NC + lax.axis_index("c")
        base = wid * b_per_w
        pltpu.sync_copy(idx_hbm.at[wid], idx_v)
        bufs = (buf0, buf1)
        sems = (sem0, sem1)
        copies = [None, None]
        copies[0] = pltpu.async_copy(table_hbm.at[idx_v.at[0]], buf0, sem0)
        if nchunk > 1:
            copies[1] = pltpu.async_copy(table_hbm.at[idx_v.at[1]], buf1, sem1)
        for c in range(nchunk):
            copies[c % 2].wait()
            pltpu.sync_copy(bufs[c % 2], out_hbm.at[pl.ds(base + c * _CH, _CH)])
            if c + 2 < nchunk:
                copies[c % 2] = pltpu.async_copy(
                    table_hbm.at[idx_v.at[c + 2]], bufs[c % 2], sems[c % 2])

    return k(word_table, idx3)


def _tc_body(g_ref, pos_ref, tt_ref, tte_ref, gam_ref, bet_ref, w_ref, b_ref, o_ref):
    x = g_ref[...] + pos_ref[...]
    tt = tt_ref[0]
    t0 = tte_ref[0:1, :]
    t1 = tte_ref[1:2, :]
    x = x + jnp.where(tt == 0, t0, t1)
    mu = jnp.mean(x, axis=1, keepdims=True)
    xc = x - mu
    var = jnp.mean(xc * xc, axis=1, keepdims=True)
    xn = xc * lax.rsqrt(var + _EPS)
    xn = xn * gam_ref[...] + bet_ref[...]
    y = lax.dot_general(xn, w_ref[...], (((1,), (1,)), ((), ())),
                        preferred_element_type=jnp.float32)
    o_ref[...] = y + b_ref[...]


def _tc_call(gathered, pos_slice, tt3, type_table, gamma2, beta2, W, b2):
    grid = _S // _BS
    return pl.pallas_call(
        _tc_body,
        grid=(grid,),
        in_specs=[
            pl.BlockSpec((_BS, _D), lambda i: (i, 0)),
            pl.BlockSpec((_BS, _D), lambda i: (i, 0)),
            pl.BlockSpec((1, _BS, 1), lambda i: (i, 0, 0)),
            pl.BlockSpec((2, _D), lambda i: (0, 0)),
            pl.BlockSpec((1, _D), lambda i: (0, 0)),
            pl.BlockSpec((1, _D), lambda i: (0, 0)),
            pl.BlockSpec((_D, _D), lambda i: (0, 0)),
            pl.BlockSpec((1, _D), lambda i: (0, 0)),
        ],
        out_specs=pl.BlockSpec((_BS, _D), lambda i: (i, 0)),
        out_shape=jax.ShapeDtypeStruct((_S, _D), jnp.float32),
    )(gathered, pos_slice, tt3, type_table, gamma2, beta2, W, b2)


def kernel(input_ids, token_type_ids, word_table, pos_table, type_table,
           ln_gamma, ln_beta, W, b):
    idx3 = input_ids.astype(jnp.int32).reshape(_NW, -1, _CH)
    gathered = _sc_gather(word_table, idx3)
    pos_slice = pos_table[:_S]
    tt3 = token_type_ids.astype(jnp.int32).reshape(_S // _BS, _BS, 1)
    out = _tc_call(gathered, pos_slice, tt3, type_table,
                   ln_gamma.reshape(1, _D), ln_beta.reshape(1, _D), W,
                   b.reshape(1, _D))
    return out.reshape(1, _S, _D)


# speedup vs baseline: 1.9529x; 1.0078x over previous
import functools

import jax
import jax.numpy as jnp
from jax import lax
from jax.experimental import pallas as pl
from jax.experimental.pallas import tpu as pltpu

try:
    from jax.experimental.pallas import tpu_sc as plsc
    _INFO = plsc.get_sparse_core_info()
    _NC, _NS = _INFO.num_cores, _INFO.num_subcores
except Exception:
    plsc = None
    _NC, _NS = 2, 16

_NW = _NC * _NS
_S = 8192
_D = 768
_EPS = 1e-12
_CH = 64
_BS = 512


def _sc_gather(word_table, idx3):
    nchunk = idx3.shape[1]
    b_per_w = nchunk * _CH
    mesh = plsc.VectorSubcoreMesh(core_axis_name="c", subcore_axis_name="s")

    @functools.partial(
        pl.kernel,
        mesh=mesh,
        out_type=jax.ShapeDtypeStruct((_S, _D), jnp.float32),
        scratch_types=[
            pltpu.VMEM((nchunk, _CH), jnp.int32),
            pltpu.VMEM((_CH, _D), jnp.float32),
            pltpu.VMEM((_CH, _D), jnp.float32),
            pltpu.SemaphoreType.DMA,
            pltpu.SemaphoreType.DMA,
        ],
    )
    def k(table_hbm, idx_hbm, out_hbm, idx_v, buf0, buf1, sem0, sem1):
        wid = lax.axis_index("s") * _NC + lax.axis_index("c")
        base = wid * b_per_w
        pltpu.sync_copy(idx_hbm.at[wid], idx_v)
        bufs = (buf0, buf1)
        sems = (sem0, sem1)
        copies = [None, None]
        copies[0] = pltpu.async_copy(table_hbm.at[idx_v.at[0]], buf0, sem0)
        if nchunk > 1:
            copies[1] = pltpu.async_copy(table_hbm.at[idx_v.at[1]], buf1, sem1)
        for c in range(nchunk):
            copies[c % 2].wait()
            pltpu.sync_copy(bufs[c % 2], out_hbm.at[pl.ds(base + c * _CH, _CH)])
            if c + 2 < nchunk:
                copies[c % 2] = pltpu.async_copy(
                    table_hbm.at[idx_v.at[c + 2]], bufs[c % 2], sems[c % 2])

    return k(word_table, idx3)


def _tc_body(g_ref, pos_ref, tt_ref, tte_ref, gam_ref, bet_ref, w_ref, b_ref, o_ref):
    x = g_ref[...] + pos_ref[...]
    tt = tt_ref[0]
    t0 = tte_ref[0:1, :]
    t1 = tte_ref[1:2, :]
    x = x + jnp.where(tt == 0, t0, t1)
    mu = jnp.mean(x, axis=1, keepdims=True)
    xc = x - mu
    var = jnp.mean(xc * xc, axis=1, keepdims=True)
    xn = xc * lax.rsqrt(var + _EPS)
    xn = xn * gam_ref[...] + bet_ref[...]
    y = lax.dot_general(xn.astype(jnp.bfloat16), w_ref[...],
                        (((1,), (1,)), ((), ())),
                        preferred_element_type=jnp.float32)
    o_ref[...] = y + b_ref[...]


def _tc_call(gathered, pos_slice, tt3, type_table, gamma2, beta2, W, b2):
    grid = _S // _BS
    return pl.pallas_call(
        _tc_body,
        grid=(grid,),
        in_specs=[
            pl.BlockSpec((_BS, _D), lambda i: (i, 0)),
            pl.BlockSpec((_BS, _D), lambda i: (i, 0)),
            pl.BlockSpec((1, _BS, 1), lambda i: (i, 0, 0)),
            pl.BlockSpec((2, _D), lambda i: (0, 0)),
            pl.BlockSpec((1, _D), lambda i: (0, 0)),
            pl.BlockSpec((1, _D), lambda i: (0, 0)),
            pl.BlockSpec((_D, _D), lambda i: (0, 0)),
            pl.BlockSpec((1, _D), lambda i: (0, 0)),
        ],
        out_specs=pl.BlockSpec((_BS, _D), lambda i: (i, 0)),
        out_shape=jax.ShapeDtypeStruct((_S, _D), jnp.float32),
    )(gathered, pos_slice, tt3, type_table, gamma2, beta2, W, b2)


def kernel(input_ids, token_type_ids, word_table, pos_table, type_table,
           ln_gamma, ln_beta, W, b):
    idx3 = input_ids.astype(jnp.int32).reshape(_NW, -1, _CH)
    gathered = _sc_gather(word_table, idx3)
    pos_slice = pos_table[:_S]
    tt3 = token_type_ids.astype(jnp.int32).reshape(_S // _BS, _BS, 1)
    out = _tc_call(gathered, pos_slice, tt3, type_table,
                   ln_gamma.reshape(1, _D), ln_beta.reshape(1, _D),
                   W.astype(jnp.bfloat16), b.reshape(1, _D))
    return out.reshape(1, _S, _D)


# speedup vs baseline: 2.0450x; 1.0471x over previous
import functools

import jax
import jax.numpy as jnp
from jax import lax
from jax.experimental import pallas as pl
from jax.experimental.pallas import tpu as pltpu

try:
    from jax.experimental.pallas import tpu_sc as plsc
    _INFO = plsc.get_sparse_core_info()
    _NC, _NS = _INFO.num_cores, _INFO.num_subcores
except Exception:
    plsc = None
    _NC, _NS = 2, 16

_NW = _NC * _NS
_S = 8192
_D = 768
_EPS = 1e-12
_CH = 64
_BS = 1024


def _sc_gather(word_table, idx3):
    nchunk = idx3.shape[1]
    b_per_w = nchunk * _CH
    mesh = plsc.VectorSubcoreMesh(core_axis_name="c", subcore_axis_name="s")

    @functools.partial(
        pl.kernel,
        mesh=mesh,
        out_type=jax.ShapeDtypeStruct((_S, _D), jnp.float32),
        scratch_types=[
            pltpu.VMEM((nchunk, _CH), jnp.int32),
            pltpu.VMEM((_CH, _D), jnp.float32),
            pltpu.VMEM((_CH, _D), jnp.float32),
            pltpu.SemaphoreType.DMA,
            pltpu.SemaphoreType.DMA,
        ],
    )
    def k(table_hbm, idx_hbm, out_hbm, idx_v, buf0, buf1, sem0, sem1):
        wid = lax.axis_index("s") * _NC + lax.axis_index("c")
        base = wid * b_per_w
        pltpu.sync_copy(idx_hbm.at[wid], idx_v)
        bufs = (buf0, buf1)
        sems = (sem0, sem1)
        copies = [None, None]
        copies[0] = pltpu.async_copy(table_hbm.at[idx_v.at[0]], buf0, sem0)
        if nchunk > 1:
            copies[1] = pltpu.async_copy(table_hbm.at[idx_v.at[1]], buf1, sem1)
        for c in range(nchunk):
            copies[c % 2].wait()
            pltpu.sync_copy(bufs[c % 2], out_hbm.at[pl.ds(base + c * _CH, _CH)])
            if c + 2 < nchunk:
                copies[c % 2] = pltpu.async_copy(
                    table_hbm.at[idx_v.at[c + 2]], bufs[c % 2], sems[c % 2])

    return k(word_table, idx3)


def _tc_body(g_ref, pos_ref, tt_ref, tte_ref, gam_ref, bet_ref, w_ref, b_ref, o_ref):
    x = g_ref[...] + pos_ref[...]
    tt = tt_ref[0]
    t0 = tte_ref[0:1, :]
    t1 = tte_ref[1:2, :]
    x = x + jnp.where(tt == 0, t0, t1)
    mu = jnp.mean(x, axis=1, keepdims=True)
    xc = x - mu
    var = jnp.mean(xc * xc, axis=1, keepdims=True)
    xn = xc * lax.rsqrt(var + _EPS)
    xn = xn * gam_ref[...] + bet_ref[...]
    y = lax.dot_general(xn.astype(jnp.bfloat16), w_ref[...],
                        (((1,), (1,)), ((), ())),
                        preferred_element_type=jnp.float32)
    o_ref[...] = y + b_ref[...]


def _tc_call(gathered, pos_slice, tt3, type_table, gamma2, beta2, W, b2):
    grid = _S // _BS
    return pl.pallas_call(
        _tc_body,
        grid=(grid,),
        in_specs=[
            pl.BlockSpec((_BS, _D), lambda i: (i, 0)),
            pl.BlockSpec((_BS, _D), lambda i: (i, 0)),
            pl.BlockSpec((1, _BS, 1), lambda i: (i, 0, 0)),
            pl.BlockSpec((2, _D), lambda i: (0, 0)),
            pl.BlockSpec((1, _D), lambda i: (0, 0)),
            pl.BlockSpec((1, _D), lambda i: (0, 0)),
            pl.BlockSpec((_D, _D), lambda i: (0, 0)),
            pl.BlockSpec((1, _D), lambda i: (0, 0)),
        ],
        out_specs=pl.BlockSpec((_BS, _D), lambda i: (i, 0)),
        out_shape=jax.ShapeDtypeStruct((_S, _D), jnp.float32),
    )(gathered, pos_slice, tt3, type_table, gamma2, beta2, W, b2)


def kernel(input_ids, token_type_ids, word_table, pos_table, type_table,
           ln_gamma, ln_beta, W, b):
    idx3 = input_ids.astype(jnp.int32).reshape(_NW, -1, _CH)
    gathered = _sc_gather(word_table, idx3)
    pos_slice = pos_table[:_S]
    tt3 = token_type_ids.astype(jnp.int32).reshape(_S // _BS, _BS, 1)
    out = _tc_call(gathered, pos_slice, tt3, type_table,
                   ln_gamma.reshape(1, _D), ln_beta.reshape(1, _D),
                   W.astype(jnp.bfloat16), b.reshape(1, _D))
    return out.reshape(1, _S, _D)
